# Initial kernel scaffold; baseline (speedup 1.0000x reference)
import jax
import jax.numpy as jnp
from jax.experimental import pallas as pl


def kernel(x, y, t, grid_ids, W1, b1, W2, b2, Wm1, bm1, Wm2, bm2, Wout):
    raise NotImplementedError("write your pallas kernel here")


# speedup vs baseline: 2.4978x; 2.4978x over previous
import functools

import jax
import jax.numpy as jnp
from jax.experimental import pallas as pl
from jax.experimental.pallas import tpu as pltpu

G, S, B, D, FF, OUT = 2, 2048, 128, 1024, 4096, 512
SBLK = 256
DBLK = 8


def _extract_kernel(wall_ref, gid_ref, x_ref, y_ref, t_ref, out_ref):
    dj = pl.program_id(0)
    si = pl.program_id(1)
    gid = gid_ref[...]
    m = gid == 0
    xb = x_ref[pl.ds(si * SBLK, SBLK), :]
    yb = y_ref[pl.ds(si * SBLK, SBLK), :]
    tb = t_ref[pl.ds(si * SBLK, SBLK), :]

    rows = []
    for i in range(DBLK):
        col = dj * DBLK + i
        w1x = jnp.where(m, wall_ref[0, col], wall_ref[4, col])
        w1y = jnp.where(m, wall_ref[1, col], wall_ref[5, col])
        w1t = jnp.where(m, wall_ref[2, col], wall_ref[6, col])
        c1 = jnp.where(m, wall_ref[3, col], wall_ref[7, col])
        w2x = jnp.where(m, wall_ref[8, col], wall_ref[12, col])
        w2y = jnp.where(m, wall_ref[9, col], wall_ref[13, col])
        w2t = jnp.where(m, wall_ref[10, col], wall_ref[14, col])
        c2 = jnp.where(m, wall_ref[11, col], wall_ref[15, col])
        a1 = xb * w1x + yb * w1y + tb * w1t + c1
        a2 = xb * w2x + yb * w2y + tb * w2t + c2
        f = jnp.tanh(a1) + jnp.maximum(a2, 0.0)
        rows.append(jnp.sum(f, axis=0, keepdims=True))
    partial = jnp.concatenate(rows, axis=0)

    @pl.when(si == 0)
    def _init():
        out_ref[...] = partial[None]

    @pl.when(si != 0)
    def _acc():
        out_ref[...] += partial[None]


def _mlp_kernel(h_ref, wm1_ref, bm1_ref, wm2_ref, bm2_ref, wout_ref, out_ref):
    h = h_ref[...] * (1.0 / S)
    h1 = jax.lax.dot_general(h, wm1_ref[...], (((0,), (0,)), ((), ())),
                             preferred_element_type=jnp.float32)
    h1 = jnp.maximum(h1 + bm1_ref[...], 0.0)
    h2 = jnp.dot(h1, wm2_ref[...],
                 preferred_element_type=jnp.float32) + bm2_ref[...]
    out_ref[...] = jnp.dot(h2, wout_ref[...],
                           preferred_element_type=jnp.float32)


@jax.jit
def kernel(x, y, t, grid_ids, W1, b1, W2, b2, Wm1, bm1, Wm2, bm2, Wout):
    wall = jnp.concatenate([
        W1[0], b1[0:1], W1[1], b1[1:2],
        W2[0], b2[0:1], W2[1], b2[1:2],
    ], axis=0)
    gid = grid_ids.reshape(1, B)

    ns, nd = S // SBLK, D // DBLK
    hsum = pl.pallas_call(
        _extract_kernel,
        grid=(nd, ns),
        in_specs=[
            pl.BlockSpec(memory_space=pltpu.SMEM),
            pl.BlockSpec((1, B), lambda dj, si: (0, 0)),
            pl.BlockSpec((S, B), lambda dj, si: (0, 0)),
            pl.BlockSpec((S, B), lambda dj, si: (0, 0)),
            pl.BlockSpec((S, B), lambda dj, si: (0, 0)),
        ],
        out_specs=pl.BlockSpec((1, DBLK, B), lambda dj, si: (dj, 0, 0)),
        out_shape=jax.ShapeDtypeStruct((nd, DBLK, B), jnp.float32),
    )(wall, gid, x, y, t)

    h_db = hsum.reshape(D, B)

    out = pl.pallas_call(
        _mlp_kernel,
        in_specs=[
            pl.BlockSpec((D, B), lambda: (0, 0)),
            pl.BlockSpec((D, FF), lambda: (0, 0)),
            pl.BlockSpec((1, FF), lambda: (0, 0)),
            pl.BlockSpec((FF, D), lambda: (0, 0)),
            pl.BlockSpec((1, D), lambda: (0, 0)),
            pl.BlockSpec((D, OUT), lambda: (0, 0)),
        ],
        out_specs=pl.BlockSpec((B, OUT), lambda: (0, 0)),
        out_shape=jax.ShapeDtypeStruct((B, OUT), jnp.float32),
    )(h_db, Wm1, bm1.reshape(1, FF), Wm2, bm2.reshape(1, D), Wout)
    return out


# speedup vs baseline: 2.5258x; 1.0112x over previous
import jax
import jax.numpy as jnp
from jax.experimental import pallas as pl
from jax.experimental.pallas import tpu as pltpu

G, S, B, D, FF, OUT = 2, 2048, 128, 1024, 4096, 512
SBLK = 256
DBLK = 8


def _fused_kernel(wall_ref, gid_ref, x_ref, y_ref, t_ref,
                  wm1_hbm, bm1_ref, wm2_hbm, bm2_ref, wout_hbm,
                  out_ref,
                  hacc_ref, wm1_v, wm2_v, wout_v, sem1, sem2, sem3):
    dj = pl.program_id(0)
    si = pl.program_id(1)
    nd = pl.num_programs(0)
    ns = pl.num_programs(1)

    @pl.when((dj == 0) & (si == 0))
    def _start_copies():
        pltpu.make_async_copy(wm1_hbm, wm1_v, sem1).start()
        pltpu.make_async_copy(wm2_hbm, wm2_v, sem2).start()
        pltpu.make_async_copy(wout_hbm, wout_v, sem3).start()

    gid = gid_ref[...]
    m = gid == 0
    xb = x_ref[pl.ds(si * SBLK, SBLK), :]
    yb = y_ref[pl.ds(si * SBLK, SBLK), :]
    tb = t_ref[pl.ds(si * SBLK, SBLK), :]

    rows = []
    for i in range(DBLK):
        col = dj * DBLK + i
        w1x = jnp.where(m, wall_ref[0, col], wall_ref[4, col])
        w1y = jnp.where(m, wall_ref[1, col], wall_ref[5, col])
        w1t = jnp.where(m, wall_ref[2, col], wall_ref[6, col])
        c1 = jnp.where(m, wall_ref[3, col], wall_ref[7, col])
        w2x = jnp.where(m, wall_ref[8, col], wall_ref[12, col])
        w2y = jnp.where(m, wall_ref[9, col], wall_ref[13, col])
        w2t = jnp.where(m, wall_ref[10, col], wall_ref[14, col])
        c2 = jnp.where(m, wall_ref[11, col], wall_ref[15, col])
        a1 = xb * w1x + yb * w1y + tb * w1t + c1
        a2 = xb * w2x + yb * w2y + tb * w2t + c2
        f = jnp.tanh(a1) + jnp.maximum(a2, 0.0)
        rows.append(jnp.sum(f, axis=0, keepdims=True))
    partial = jnp.concatenate(rows, axis=0)

    @pl.when(si == 0)
    def _init():
        hacc_ref[pl.ds(dj * DBLK, DBLK), :] = partial

    @pl.when(si != 0)
    def _acc():
        hacc_ref[pl.ds(dj * DBLK, DBLK), :] += partial

    @pl.when((dj == nd - 1) & (si == ns - 1))
    def _mlp():
        pltpu.make_async_copy(wm1_hbm, wm1_v, sem1).wait()
        pltpu.make_async_copy(wm2_hbm, wm2_v, sem2).wait()
        pltpu.make_async_copy(wout_hbm, wout_v, sem3).wait()
        h = hacc_ref[...] * (1.0 / S)
        h1 = jax.lax.dot_general(h, wm1_v[...], (((0,), (0,)), ((), ())),
                                 preferred_element_type=jnp.float32)
        h1 = jnp.maximum(h1 + bm1_ref[...], 0.0)
        h2 = jnp.dot(h1, wm2_v[...],
                     preferred_element_type=jnp.float32) + bm2_ref[...]
        out_ref[...] = jnp.dot(h2, wout_v[...],
                               preferred_element_type=jnp.float32)


@jax.jit
def kernel(x, y, t, grid_ids, W1, b1, W2, b2, Wm1, bm1, Wm2, bm2, Wout):
    wall = jnp.concatenate([
        W1[0], b1[0:1], W1[1], b1[1:2],
        W2[0], b2[0:1], W2[1], b2[1:2],
    ], axis=0)
    gid = grid_ids.reshape(1, B)

    ns, nd = S // SBLK, D // DBLK
    out = pl.pallas_call(
        _fused_kernel,
        grid=(nd, ns),
        in_specs=[
            pl.BlockSpec(memory_space=pltpu.SMEM),
            pl.BlockSpec((1, B), lambda dj, si: (0, 0)),
            pl.BlockSpec((S, B), lambda dj, si: (0, 0)),
            pl.BlockSpec((S, B), lambda dj, si: (0, 0)),
            pl.BlockSpec((S, B), lambda dj, si: (0, 0)),
            pl.BlockSpec(memory_space=pl.ANY),
            pl.BlockSpec((1, FF), lambda dj, si: (0, 0)),
            pl.BlockSpec(memory_space=pl.ANY),
            pl.BlockSpec((1, D), lambda dj, si: (0, 0)),
            pl.BlockSpec(memory_space=pl.ANY),
        ],
        out_specs=pl.BlockSpec((B, OUT), lambda dj, si: (0, 0)),
        out_shape=jax.ShapeDtypeStruct((B, OUT), jnp.float32),
        scratch_shapes=[
            pltpu.VMEM((D, B), jnp.float32),
            pltpu.VMEM((D, FF), jnp.float32),
            pltpu.VMEM((FF, D), jnp.float32),
            pltpu.VMEM((D, OUT), jnp.float32),
            pltpu.SemaphoreType.DMA,
            pltpu.SemaphoreType.DMA,
            pltpu.SemaphoreType.DMA,
        ],
    )(wall, gid, x, y, t, Wm1, bm1.reshape(1, FF), Wm2, bm2.reshape(1, D),
      Wout)
    return out


# speedup vs baseline: 4.2712x; 1.6911x over previous
import jax
import jax.numpy as jnp
from jax.experimental import pallas as pl
from jax.experimental.pallas import tpu as pltpu

G, S, B, D, FF, OUT = 2, 2048, 128, 1024, 4096, 512
N = S * B
MBLK = 512


def _fused_kernel(wcat_ref, gidf_ref, xf_ref, yf_ref, tf_ref,
                  wm1_hbm, bm1_ref, wm2_hbm, bm2_ref, wout_hbm,
                  out_ref,
                  inpt_ref, hacc_ref, wm1_v, wm2_v, wout_v,
                  sem1, sem2, sem3):
    mi = pl.program_id(0)
    nm = pl.num_programs(0)

    @pl.when(mi == 0)
    def _prologue():
        pltpu.make_async_copy(wm1_hbm, wm1_v, sem1).start()
        pltpu.make_async_copy(wm2_hbm, wm2_v, sem2).start()
        pltpu.make_async_copy(wout_hbm, wout_v, sem3).start()
        m0 = jnp.where(gidf_ref[...] == 0, 1.0, 0.0)
        xr, yr, tr = xf_ref[...], yf_ref[...], tf_ref[...]
        x0 = xr * m0
        y0 = yr * m0
        t0 = tr * m0
        inpt_ref[...] = jnp.concatenate(
            [x0, xr - x0, y0, yr - y0, t0, tr - t0, m0, 1.0 - m0], axis=0)

    lhs = inpt_ref[:, pl.ds(mi * MBLK, MBLK)]
    a = jax.lax.dot_general(wcat_ref[...], lhs, (((0,), (0,)), ((), ())),
                            preferred_element_type=jnp.float32)
    f = jnp.tanh(a[:D, :]) + jnp.maximum(a[D:, :], 0.0)
    part = f[:, 0:B]
    for j in range(1, MBLK // B):
        part = part + f[:, j * B:(j + 1) * B]

    @pl.when(mi == 0)
    def _init():
        hacc_ref[...] = part

    @pl.when(mi != 0)
    def _acc():
        hacc_ref[...] += part

    @pl.when(mi == nm - 1)
    def _mlp():
        pltpu.make_async_copy(wm1_hbm, wm1_v, sem1).wait()
        pltpu.make_async_copy(wm2_hbm, wm2_v, sem2).wait()
        pltpu.make_async_copy(wout_hbm, wout_v, sem3).wait()
        h = hacc_ref[...] * (1.0 / S)
        h1 = jax.lax.dot_general(h, wm1_v[...], (((0,), (0,)), ((), ())),
                                 preferred_element_type=jnp.float32)
        h1 = jnp.maximum(h1 + bm1_ref[...], 0.0)
        h2 = jnp.dot(h1, wm2_v[...],
                     preferred_element_type=jnp.float32) + bm2_ref[...]
        out_ref[...] = jnp.dot(h2, wout_v[...],
                               preferred_element_type=jnp.float32)


@jax.jit
def kernel(x, y, t, grid_ids, W1, b1, W2, b2, Wm1, bm1, Wm2, bm2, Wout):
    top = jnp.stack([W1[0, 0], W1[1, 0], W1[0, 1], W1[1, 1],
                     W1[0, 2], W1[1, 2], b1[0], b1[1]])
    bot = jnp.stack([W2[0, 0], W2[1, 0], W2[0, 1], W2[1, 1],
                     W2[0, 2], W2[1, 2], b2[0], b2[1]])
    wcat = jnp.concatenate([top, bot], axis=1)

    xf = x.reshape(1, N)
    yf = y.reshape(1, N)
    tf = t.reshape(1, N)
    gidf = jnp.tile(grid_ids, S).reshape(1, N)

    nm = N // MBLK
    out = pl.pallas_call(
        _fused_kernel,
        grid=(nm,),
        in_specs=[
            pl.BlockSpec((8, 2 * D), lambda mi: (0, 0)),
            pl.BlockSpec((1, N), lambda mi: (0, 0)),
            pl.BlockSpec((1, N), lambda mi: (0, 0)),
            pl.BlockSpec((1, N), lambda mi: (0, 0)),
            pl.BlockSpec((1, N), lambda mi: (0, 0)),
            pl.BlockSpec(memory_space=pl.ANY),
            pl.BlockSpec((1, FF), lambda mi: (0, 0)),
            pl.BlockSpec(memory_space=pl.ANY),
            pl.BlockSpec((1, D), lambda mi: (0, 0)),
            pl.BlockSpec(memory_space=pl.ANY),
        ],
        out_specs=pl.BlockSpec((B, OUT), lambda mi: (0, 0)),
        out_shape=jax.ShapeDtypeStruct((B, OUT), jnp.float32),
        scratch_shapes=[
            pltpu.VMEM((8, N), jnp.float32),
            pltpu.VMEM((D, B), jnp.float32),
            pltpu.VMEM((D, FF), jnp.float32),
            pltpu.VMEM((FF, D), jnp.float32),
            pltpu.VMEM((D, OUT), jnp.float32),
            pltpu.SemaphoreType.DMA,
            pltpu.SemaphoreType.DMA,
            pltpu.SemaphoreType.DMA,
        ],
    )(wcat, gidf, xf, yf, tf, Wm1, bm1.reshape(1, FF), Wm2,
      bm2.reshape(1, D), Wout)
    return out


# speedup vs baseline: 4.3628x; 1.0214x over previous
import jax
import jax.numpy as jnp
from jax.experimental import pallas as pl
from jax.experimental.pallas import tpu as pltpu

G, S, B, D, FF, OUT = 2, 2048, 128, 1024, 4096, 512
N = S * B
MBLK = 512


def _fused_kernel(wcat_ref, gidf_ref, xf_ref, yf_ref, tf_ref,
                  wm1_hbm, bm1_ref, wm2_hbm, bm2_ref, wout_hbm,
                  out_ref,
                  inpt_ref, hacc_ref, wm1_v, wm2_v, wout_v,
                  sem1, sem2, sem3):
    mi = pl.program_id(0)
    nm = pl.num_programs(0)

    @pl.when(mi == 0)
    def _prologue():
        pltpu.make_async_copy(wm1_hbm, wm1_v, sem1).start()
        pltpu.make_async_copy(wm2_hbm, wm2_v, sem2).start()
        pltpu.make_async_copy(wout_hbm, wout_v, sem3).start()
        m0 = jnp.where(gidf_ref[...] == 0, 1.0, 0.0)
        xr, yr, tr = xf_ref[...], yf_ref[...], tf_ref[...]
        x0 = xr * m0
        y0 = yr * m0
        t0 = tr * m0
        inpt_ref[...] = jnp.concatenate(
            [x0, xr - x0, y0, yr - y0, t0, tr - t0, m0, 1.0 - m0],
            axis=0).astype(jnp.bfloat16)

    lhs = inpt_ref[:, pl.ds(mi * MBLK, MBLK)]
    a = jax.lax.dot_general(wcat_ref[...], lhs, (((0,), (0,)), ((), ())),
                            preferred_element_type=jnp.float32)
    f = jnp.tanh(a[:D, :]) + jnp.maximum(a[D:, :], 0.0)
    part = f[:, 0:B]
    for j in range(1, MBLK // B):
        part = part + f[:, j * B:(j + 1) * B]

    @pl.when(mi == 0)
    def _init():
        hacc_ref[...] = part

    @pl.when(mi != 0)
    def _acc():
        hacc_ref[...] += part

    @pl.when(mi == nm - 1)
    def _mlp():
        pltpu.make_async_copy(wm1_hbm, wm1_v, sem1).wait()
        pltpu.make_async_copy(wm2_hbm, wm2_v, sem2).wait()
        pltpu.make_async_copy(wout_hbm, wout_v, sem3).wait()
        h = hacc_ref[...] * (1.0 / S)
        h1 = jax.lax.dot_general(h, wm1_v[...], (((0,), (0,)), ((), ())),
                                 preferred_element_type=jnp.float32)
        h1 = jnp.maximum(h1 + bm1_ref[...], 0.0)
        h2 = jnp.dot(h1, wm2_v[...],
                     preferred_element_type=jnp.float32) + bm2_ref[...]
        out_ref[...] = jnp.dot(h2, wout_v[...],
                               preferred_element_type=jnp.float32)


@jax.jit
def kernel(x, y, t, grid_ids, W1, b1, W2, b2, Wm1, bm1, Wm2, bm2, Wout):
    top = jnp.stack([W1[0, 0], W1[1, 0], W1[0, 1], W1[1, 1],
                     W1[0, 2], W1[1, 2], b1[0], b1[1]])
    bot = jnp.stack([W2[0, 0], W2[1, 0], W2[0, 1], W2[1, 1],
                     W2[0, 2], W2[1, 2], b2[0], b2[1]])
    wcat = jnp.concatenate([top, bot], axis=1).astype(jnp.bfloat16)

    xf = x.reshape(1, N)
    yf = y.reshape(1, N)
    tf = t.reshape(1, N)
    gidf = jnp.tile(grid_ids, S).reshape(1, N)

    nm = N // MBLK
    out = pl.pallas_call(
        _fused_kernel,
        grid=(nm,),
        in_specs=[
            pl.BlockSpec((8, 2 * D), lambda mi: (0, 0)),
            pl.BlockSpec((1, N), lambda mi: (0, 0)),
            pl.BlockSpec((1, N), lambda mi: (0, 0)),
            pl.BlockSpec((1, N), lambda mi: (0, 0)),
            pl.BlockSpec((1, N), lambda mi: (0, 0)),
            pl.BlockSpec(memory_space=pl.ANY),
            pl.BlockSpec((1, FF), lambda mi: (0, 0)),
            pl.BlockSpec(memory_space=pl.ANY),
            pl.BlockSpec((1, D), lambda mi: (0, 0)),
            pl.BlockSpec(memory_space=pl.ANY),
        ],
        out_specs=pl.BlockSpec((B, OUT), lambda mi: (0, 0)),
        out_shape=jax.ShapeDtypeStruct((B, OUT), jnp.float32),
        scratch_shapes=[
            pltpu.VMEM((8, N), jnp.bfloat16),
            pltpu.VMEM((D, B), jnp.float32),
            pltpu.VMEM((D, FF), jnp.float32),
            pltpu.VMEM((FF, D), jnp.float32),
            pltpu.VMEM((D, OUT), jnp.float32),
            pltpu.SemaphoreType.DMA,
            pltpu.SemaphoreType.DMA,
            pltpu.SemaphoreType.DMA,
        ],
    )(wcat, gidf, xf, yf, tf, Wm1, bm1.reshape(1, FF), Wm2,
      bm2.reshape(1, D), Wout)
    return out


# speedup vs baseline: 5.2120x; 1.1946x over previous
import jax
import jax.numpy as jnp
from jax.experimental import pallas as pl
from jax.experimental.pallas import tpu as pltpu

G, S, B, D, FF, OUT = 2, 2048, 128, 1024, 4096, 512
N = S * B
MBLK = 1024


def _fused_kernel(wcat_ref, gidf_ref, xf_ref, yf_ref, tf_ref,
                  wm1_hbm, bm1_ref, wm2_hbm, bm2_ref, wout_hbm,
                  out_ref,
                  inpt_ref, hacc_ref, wm1_v, wm2_v, wout_v,
                  sem1, sem2, sem3):
    mi = pl.program_id(0)
    nm = pl.num_programs(0)

    @pl.when(mi == 0)
    def _prologue():
        pltpu.make_async_copy(wm1_hbm, wm1_v, sem1).start()
        pltpu.make_async_copy(wm2_hbm, wm2_v, sem2).start()
        pltpu.make_async_copy(wout_hbm, wout_v, sem3).start()
        m0 = jnp.where(gidf_ref[...] == 0, 1.0, 0.0)
        xr, yr, tr = xf_ref[...], yf_ref[...], tf_ref[...]
        x0 = xr * m0
        y0 = yr * m0
        t0 = tr * m0
        inpt_ref[...] = jnp.concatenate(
            [x0, xr - x0, y0, yr - y0, t0, tr - t0, m0, 1.0 - m0],
            axis=0).astype(jnp.bfloat16)

    lhs = inpt_ref[:, pl.ds(mi * MBLK, MBLK)]
    a = jax.lax.dot_general(wcat_ref[...], lhs, (((0,), (0,)), ((), ())),
                            preferred_element_type=jnp.float32)
    f = jnp.tanh(a[:D, :]) + jnp.maximum(a[D:, :], 0.0)
    part = f[:, 0:B]
    for j in range(1, MBLK // B):
        part = part + f[:, j * B:(j + 1) * B]

    @pl.when(mi == 0)
    def _init():
        hacc_ref[...] = part

    @pl.when(mi != 0)
    def _acc():
        hacc_ref[...] += part

    @pl.when(mi == nm - 1)
    def _mlp():
        pltpu.make_async_copy(wm1_hbm, wm1_v, sem1).wait()
        pltpu.make_async_copy(wm2_hbm, wm2_v, sem2).wait()
        pltpu.make_async_copy(wout_hbm, wout_v, sem3).wait()
        h = hacc_ref[...] * (1.0 / S)
        h1 = jax.lax.dot_general(h, wm1_v[...], (((0,), (0,)), ((), ())),
                                 preferred_element_type=jnp.float32)
        h1 = jnp.maximum(h1 + bm1_ref[...], 0.0)
        h2 = jnp.dot(h1, wm2_v[...],
                     preferred_element_type=jnp.float32) + bm2_ref[...]
        out_ref[...] = jnp.dot(h2, wout_v[...],
                               preferred_element_type=jnp.float32)


@jax.jit
def kernel(x, y, t, grid_ids, W1, b1, W2, b2, Wm1, bm1, Wm2, bm2, Wout):
    top = jnp.stack([W1[0, 0], W1[1, 0], W1[0, 1], W1[1, 1],
                     W1[0, 2], W1[1, 2], b1[0], b1[1]])
    bot = jnp.stack([W2[0, 0], W2[1, 0], W2[0, 1], W2[1, 1],
                     W2[0, 2], W2[1, 2], b2[0], b2[1]])
    wcat = jnp.concatenate([top, bot], axis=1).astype(jnp.bfloat16)

    xf = x.reshape(1, N)
    yf = y.reshape(1, N)
    tf = t.reshape(1, N)
    gidf = jnp.tile(grid_ids, S).reshape(1, N)

    nm = N // MBLK
    out = pl.pallas_call(
        _fused_kernel,
        grid=(nm,),
        in_specs=[
            pl.BlockSpec((8, 2 * D), lambda mi: (0, 0)),
            pl.BlockSpec((1, N), lambda mi: (0, 0)),
            pl.BlockSpec((1, N), lambda mi: (0, 0)),
            pl.BlockSpec((1, N), lambda mi: (0, 0)),
            pl.BlockSpec((1, N), lambda mi: (0, 0)),
            pl.BlockSpec(memory_space=pl.ANY),
            pl.BlockSpec((1, FF), lambda mi: (0, 0)),
            pl.BlockSpec(memory_space=pl.ANY),
            pl.BlockSpec((1, D), lambda mi: (0, 0)),
            pl.BlockSpec(memory_space=pl.ANY),
        ],
        out_specs=pl.BlockSpec((B, OUT), lambda mi: (0, 0)),
        out_shape=jax.ShapeDtypeStruct((B, OUT), jnp.float32),
        scratch_shapes=[
            pltpu.VMEM((8, N), jnp.bfloat16),
            pltpu.VMEM((D, B), jnp.float32),
            pltpu.VMEM((D, FF), jnp.float32),
            pltpu.VMEM((FF, D), jnp.float32),
            pltpu.VMEM((D, OUT), jnp.float32),
            pltpu.SemaphoreType.DMA,
            pltpu.SemaphoreType.DMA,
            pltpu.SemaphoreType.DMA,
        ],
    )(wcat, gidf, xf, yf, tf, Wm1, bm1.reshape(1, FF), Wm2,
      bm2.reshape(1, D), Wout)
    return out
